# Initial kernel scaffold; baseline (speedup 1.0000x reference)
import jax
import jax.numpy as jnp
from jax.experimental import pallas as pl


def kernel(feats, edge_index, params):
    raise NotImplementedError("write your pallas kernel here")


# speedup vs baseline: 14.7961x; 14.7961x over previous
import functools

import jax
import jax.numpy as jnp
from jax import lax
from jax.experimental import pallas as pl
from jax.experimental.pallas import tpu as pltpu
from jax.experimental.pallas import tpu_sc as plsc

N = 10000
E = 320000
D_IN = 128
HID = 64
H2 = 128
H4 = 256
HEADS = 4
HEAD_DIM = 32
N_LAYER = 2

NC = 2
NS = 16
NW = NC * NS
CH = 80
EPW = E // NW
NCHUNK = EPW // CH
RPT = 1000
NZT = N // RPT

@functools.cache
def _mesh():
    return plsc.VectorSubcoreMesh(core_axis_name="c", subcore_axis_name="s")


def _segsum_body(x_hbm, sidx_hbm, didx_hbm, zeros_hbm, out_hbm,
                 sidx_v, didx_v, rows_v, acc_sh, sem):
    cid = lax.axis_index("c")
    sid = lax.axis_index("s")
    wid = sid * NC + cid
    r0 = sid * RPT

    @pl.when(sid < NZT)
    def _():
        pltpu.sync_copy(zeros_hbm.at[pl.ds(r0, RPT)],
                        acc_sh.at[pl.ds(r0, RPT)])

    pltpu.sync_copy(sidx_hbm.at[wid], sidx_v)
    pltpu.sync_copy(didx_hbm.at[wid], didx_v)
    plsc.subcore_barrier()

    def body(j, carry):
        pltpu.async_copy(x_hbm.at[sidx_v.at[j]], rows_v, sem).wait()
        pltpu.sync_copy(rows_v, acc_sh.at[didx_v.at[j]], add=True)
        return carry

    lax.fori_loop(0, NCHUNK, body, 0)
    plsc.subcore_barrier()

    @pl.when(sid < NZT)
    def _():
        pltpu.sync_copy(acc_sh.at[pl.ds(r0, RPT)],
                        out_hbm.at[cid, pl.ds(r0, RPT)])


@functools.cache
def _segsum_k():
    return pl.kernel(
        _segsum_body,
        out_type=jax.ShapeDtypeStruct((NC, N, H2), jnp.float32),
        mesh=_mesh(),
        scratch_types=[
            pltpu.VMEM((NCHUNK, CH), jnp.int32),
            pltpu.VMEM((NCHUNK, CH), jnp.int32),
            pltpu.VMEM((CH, H2), jnp.float32),
            pltpu.VMEM_SHARED((N, H2), jnp.float32),
            pltpu.SemaphoreType.DMA,
        ],
    )


def _segsum(*args):
    return _segsum_k()(*args)


def _gat_gather_body(hw_hbm, erp_hbm, sidx_hbm, didx_hbm,
                     hg_hbm, erg_hbm,
                     sidx_v, didx_v, rh_v, rb_v, sem0, sem1):
    cid = lax.axis_index("c")
    sid = lax.axis_index("s")
    wid = sid * NC + cid
    pltpu.sync_copy(sidx_hbm.at[wid], sidx_v)
    pltpu.sync_copy(didx_hbm.at[wid], didx_v)

    def body(j, carry):
        base = wid * EPW + j * CH
        a = pltpu.async_copy(hw_hbm.at[sidx_v.at[j]], rh_v, sem0)
        b = pltpu.async_copy(erp_hbm.at[didx_v.at[j]], rb_v, sem1)
        a.wait()
        b.wait()
        pltpu.sync_copy(rh_v, hg_hbm.at[pl.ds(base, CH)])
        pltpu.sync_copy(rb_v, erg_hbm.at[pl.ds(base, CH)])
        return carry

    lax.fori_loop(0, NCHUNK, body, 0)


@functools.cache
def _gat_gather_k():
    return pl.kernel(
        _gat_gather_body,
        out_type=(jax.ShapeDtypeStruct((E, H2), jnp.float32),
                  jax.ShapeDtypeStruct((E, H2), jnp.float32)),
        mesh=_mesh(),
        scratch_types=[
            pltpu.VMEM((NCHUNK, CH), jnp.int32),
            pltpu.VMEM((NCHUNK, CH), jnp.int32),
            pltpu.VMEM((CH, H2), jnp.float32),
            pltpu.VMEM((CH, H2), jnp.float32),
            pltpu.SemaphoreType.DMA,
            pltpu.SemaphoreType.DMA,
        ],
    )


def _gat_gather(*args):
    return _gat_gather_k()(*args)


def _escatter_body(vals_hbm, didx_hbm, zeros_hbm, out_hbm,
                   didx_v, rows_v, acc_sh):
    cid = lax.axis_index("c")
    sid = lax.axis_index("s")
    wid = sid * NC + cid
    r0 = sid * RPT

    @pl.when(sid < NZT)
    def _():
        pltpu.sync_copy(zeros_hbm.at[pl.ds(r0, RPT)],
                        acc_sh.at[pl.ds(r0, RPT)])

    pltpu.sync_copy(didx_hbm.at[wid], didx_v)
    plsc.subcore_barrier()

    def body(j, carry):
        base = wid * EPW + j * CH
        pltpu.sync_copy(vals_hbm.at[pl.ds(base, CH)], rows_v)
        pltpu.sync_copy(rows_v, acc_sh.at[didx_v.at[j]], add=True)
        return carry

    lax.fori_loop(0, NCHUNK, body, 0)
    plsc.subcore_barrier()

    @pl.when(sid < NZT)
    def _():
        pltpu.sync_copy(acc_sh.at[pl.ds(r0, RPT)],
                        out_hbm.at[cid, pl.ds(r0, RPT)])


@functools.cache
def _escatter_k():
    return pl.kernel(
        _escatter_body,
        out_type=jax.ShapeDtypeStruct((NC, N, H2), jnp.float32),
        mesh=_mesh(),
        scratch_types=[
            pltpu.VMEM((NCHUNK, CH), jnp.int32),
            pltpu.VMEM((CH, H2), jnp.float32),
            pltpu.VMEM_SHARED((N, H2), jnp.float32),
        ],
    )


def _escatter(*args):
    return _escatter_k()(*args)


RB = 2000
GRID_N = N // RB
EB = 5000
GRID_E = E // EB

PADW = H2 - HID


def _full(shape):
    return pl.BlockSpec(shape, lambda i: tuple(0 for _ in shape))


def _rows(shape):
    return pl.BlockSpec(shape, lambda i: (i,) + tuple(0 for _ in shape[1:]))


def _pad128(x):
    return jnp.concatenate(
        [x, jnp.zeros((x.shape[0], PADW), jnp.float32)], axis=1)


def _norms_body(ds_ref, dd_ref, ns_ref, nd_ref):
    ds = (ds_ref[0] + ds_ref[1])[:, :1]
    dd = (dd_ref[0] + dd_ref[1])[:, :1]
    ns_ref[...] = lax.rsqrt(jnp.maximum(ds, 1.0))
    nd_ref[...] = lax.rsqrt(jnp.maximum(dd, 1.0))


def _t_norms(dsrc, ddst):
    return pl.pallas_call(
        _norms_body,
        out_shape=(jax.ShapeDtypeStruct((N, 1), jnp.float32),
                   jax.ShapeDtypeStruct((N, 1), jnp.float32)),
        grid=(GRID_N,),
        in_specs=[pl.BlockSpec((NC, RB, H2), lambda i: (0, i, 0)),
                  pl.BlockSpec((NC, RB, H2), lambda i: (0, i, 0))],
        out_specs=(_rows((RB, 1)), _rows((RB, 1))),
    )(dsrc, ddst)


def _t0_body(x_ref, ns_ref, w_ref, o_ref):
    h = jnp.dot(x_ref[...] * ns_ref[...], w_ref[...],
                preferred_element_type=jnp.float32)
    o_ref[...] = _pad128(h)


def _t0(x, ns, w):
    return pl.pallas_call(
        _t0_body,
        out_shape=jax.ShapeDtypeStruct((N, H2), jnp.float32),
        grid=(GRID_N,),
        in_specs=[_rows((RB, x.shape[1])), _rows((RB, 1)),
                  _full(w.shape)],
        out_specs=_rows((RB, H2)),
    )(x, ns, w)


def _t1_body(m_ref, nd_ref, ns_ref, b_ref, w_ref, o_ref):
    m = (m_ref[0] + m_ref[1])[:, :HID]
    x = jnp.maximum(m * nd_ref[...] + b_ref[...], 0.0)
    h = jnp.dot(x * ns_ref[...], w_ref[...],
                preferred_element_type=jnp.float32)
    o_ref[...] = _pad128(h)


def _t1(m, nd, ns, b, w):
    return pl.pallas_call(
        _t1_body,
        out_shape=jax.ShapeDtypeStruct((N, H2), jnp.float32),
        grid=(GRID_N,),
        in_specs=[pl.BlockSpec((NC, RB, H2), lambda i: (0, i, 0)),
                  _rows((RB, 1)), _rows((RB, 1)), _full((1, HID)),
                  _full(w.shape)],
        out_specs=_rows((RB, H2)),
    )(m, nd, ns, b, w)


def _t2_body(m_ref, nd_ref, ns_ref, b_ref, o_ref):
    m = (m_ref[0] + m_ref[1])[:, :HID]
    x = jnp.maximum(m * nd_ref[...] + b_ref[...], 0.0)
    o_ref[...] = _pad128(x * ns_ref[...])


def _t2(m, nd, ns, b):
    return pl.pallas_call(
        _t2_body,
        out_shape=jax.ShapeDtypeStruct((N, H2), jnp.float32),
        grid=(GRID_N,),
        in_specs=[pl.BlockSpec((NC, RB, H2), lambda i: (0, i, 0)),
                  _rows((RB, 1)), _rows((RB, 1)), _full((1, HID))],
        out_specs=_rows((RB, H2)),
    )(m, nd, ns, b)


def _t3_body(m_ref, w_ref, nd_ref, b_ref, o_ref):
    m = (m_ref[0] + m_ref[1])[:, :HID]
    mm = jnp.dot(m, w_ref[...], preferred_element_type=jnp.float32)
    o_ref[...] = jnp.maximum(mm * nd_ref[...] + b_ref[...], 0.0)


def _t3(m, w, nd, b):
    return pl.pallas_call(
        _t3_body,
        out_shape=jax.ShapeDtypeStruct((N, w.shape[1]), jnp.float32),
        grid=(GRID_N,),
        in_specs=[pl.BlockSpec((NC, RB, H2), lambda i: (0, i, 0)),
                  _full(w.shape), _rows((RB, 1)), _full((1, w.shape[1]))],
        out_specs=_rows((RB, w.shape[1])),
    )(m, w, nd, b)


def _t4_body(h_ref, w_ref, al_ref, ar_ref, hw_ref, el_ref, erp_ref):
    hw = jnp.dot(h_ref[...], w_ref[...], preferred_element_type=jnp.float32)
    hw_ref[...] = hw
    el_ref[...] = (hw * al_ref[...]).reshape(RB, HEADS, HEAD_DIM).sum(-1)
    er = (hw * ar_ref[...]).reshape(RB, HEADS, HEAD_DIM).sum(-1)
    erp_ref[...] = jnp.concatenate(
        [er, jnp.zeros((RB, H2 - HEADS), jnp.float32)], axis=1)


def _t4(h, w, alf, arf):
    return pl.pallas_call(
        _t4_body,
        out_shape=(jax.ShapeDtypeStruct((N, H2), jnp.float32),
                   jax.ShapeDtypeStruct((N, HEADS), jnp.float32),
                   jax.ShapeDtypeStruct((N, H2), jnp.float32)),
        grid=(GRID_N,),
        in_specs=[_rows((RB, H2)), _full((H2, H2)),
                  _full((1, H2)), _full((1, H2))],
        out_specs=(_rows((RB, H2)), _rows((RB, HEADS)), _rows((RB, H2))),
    )(h, w, alf, arf)


def _t5_body(el_ref, erp_ref, c_ref):
    c = jnp.max(el_ref[...], axis=0) + jnp.max(erp_ref[...][:, :HEADS], axis=0)
    c_ref[...] = jnp.maximum(c, 0.0)[None]


def _t5(el, erp):
    return pl.pallas_call(
        _t5_body,
        out_shape=jax.ShapeDtypeStruct((1, HEADS), jnp.float32),
        grid=(1,),
        in_specs=[_full((N, HEADS)), _full((N, H2))],
        out_specs=_full((1, HEADS)),
    )(el, erp)


def _t6_body(hg_ref, erg_ref, c_ref, al_ref, ee_ref, hs_ref):
    hg = hg_ref[...]
    elg = (hg * al_ref[...]).reshape(EB, HEADS, HEAD_DIM).sum(-1)
    s = elg + erg_ref[...][:, :HEADS]
    e = jnp.maximum(s, 0.2 * s)
    ee = jnp.exp(e - c_ref[...])
    ee_ref[...] = jnp.concatenate(
        [ee, jnp.zeros((EB, H2 - HEADS), jnp.float32)], axis=1)
    ee128 = jnp.broadcast_to(ee[:, :, None], (EB, HEADS, HEAD_DIM))
    hs_ref[...] = hg * ee128.reshape(EB, H2)


def _t6(hg, erg, c, alf):
    return pl.pallas_call(
        _t6_body,
        out_shape=(jax.ShapeDtypeStruct((E, H2), jnp.float32),
                   jax.ShapeDtypeStruct((E, H2), jnp.float32)),
        grid=(GRID_E,),
        in_specs=[_rows((EB, H2)), _rows((EB, H2)),
                  _full((1, HEADS)), _full((1, H2))],
        out_specs=(_rows((EB, H2)), _rows((EB, H2))),
    )(hg, erg, c, alf)


def _t7_body(pre_ref, es_ref, b_ref, ns_ref, h1_ref, h1ns_ref):
    es = (es_ref[0] + es_ref[1])[:, :HEADS]
    den = jnp.where(es > 0.0, es, 1.0)
    den128 = jnp.broadcast_to(den[:, :, None], (RB, HEADS, HEAD_DIM))
    h1 = jnp.maximum((pre_ref[0] + pre_ref[1]) / den128.reshape(RB, H2)
                     + b_ref[...], 0.0)
    h1_ref[...] = h1
    h1ns_ref[...] = h1 * ns_ref[...]


def _t7(pre, esum, b, ns):
    return pl.pallas_call(
        _t7_body,
        out_shape=(jax.ShapeDtypeStruct((N, H2), jnp.float32),
                   jax.ShapeDtypeStruct((N, H2), jnp.float32)),
        grid=(GRID_N,),
        in_specs=[pl.BlockSpec((NC, RB, H2), lambda i: (0, i, 0)),
                  pl.BlockSpec((NC, RB, H2), lambda i: (0, i, 0)),
                  _full((1, H2)), _rows((RB, 1))],
        out_specs=(_rows((RB, H2)), _rows((RB, H2))),
    )(pre, esum, b, ns)


def _t8_body(m_ref, nd_ref, wgc_ref, bgc_ref, h1_ref, hp_ref,
             lng_ref, lnb_ref, wr1_ref, wr2_ref, br_ref,
             wz1_ref, wz2_ref, bz_ref, wh1_ref, wh2_ref, bh_ref, o_ref):
    mm = jnp.dot(m_ref[0] + m_ref[1], wgc_ref[...],
                 preferred_element_type=jnp.float32)
    h1_1 = jnp.maximum(mm * nd_ref[...] + bgc_ref[...], 0.0)
    a = h1_1[:, :H2]
    g = h1_1[:, H2:]
    glu = a * jax.nn.sigmoid(g)
    h2 = (h1_ref[...] + glu) * jnp.sqrt(jnp.float32(0.5))
    mu = jnp.mean(h2, axis=-1, keepdims=True)
    var = jnp.mean((h2 - mu) ** 2, axis=-1, keepdims=True)
    h2n = (h2 - mu) / jnp.sqrt(var + 1e-5) * lng_ref[...] + lnb_ref[...]
    hp = hp_ref[...]
    dot = functools.partial(jnp.dot, preferred_element_type=jnp.float32)
    r = jax.nn.sigmoid(dot(hp, wr1_ref[...]) + dot(h2n, wr2_ref[...])
                       + br_ref[...])
    z = jax.nn.sigmoid(dot(hp, wz1_ref[...]) + dot(h2n, wz2_ref[...])
                       + bz_ref[...])
    hc = jnp.tanh(dot(h2n, wh1_ref[...]) + dot(r * hp, wh2_ref[...])
                  + bh_ref[...])
    o_ref[...] = z * hp + (1.0 - z) * hc


def _t8(m, nd, wgc, bgc, h1, hprev, lng, lnb, wr1, wr2, br,
        wz1, wz2, bz, wh1, wh2, bh):
    return pl.pallas_call(
        _t8_body,
        out_shape=jax.ShapeDtypeStruct((N, H2), jnp.float32),
        grid=(GRID_N,),
        in_specs=[pl.BlockSpec((NC, RB, H2), lambda i: (0, i, 0)),
                  _rows((RB, 1)), _full((H2, H4)), _full((1, H4)),
                  _rows((RB, H2)), _rows((RB, H2)),
                  _full((1, H2)), _full((1, H2)),
                  _full((H2, H2)), _full((H2, H2)), _full((1, H2)),
                  _full((H2, H2)), _full((H2, H2)), _full((1, H2)),
                  _full((H2, H2)), _full((H2, H2)), _full((1, H2))],
        out_specs=_rows((RB, H2)),
    )(m, nd, wgc, bgc, h1, hprev, lng, lnb, wr1, wr2, br,
      wz1, wz2, bz, wh1, wh2, bh)


def _t9_body(h_ref, o_ref):
    o_ref[...] = jnp.mean(h_ref[...], axis=0, keepdims=True)


def _t9(h):
    return pl.pallas_call(
        _t9_body,
        out_shape=jax.ShapeDtypeStruct((1, H2), jnp.float32),
        grid=(1,),
        in_specs=[_full((N, H2))],
        out_specs=_full((1, H2)),
    )(h)


def kernel(feats, edge_index, params):
    p = params
    sidx2 = edge_index[0].reshape(NW, NCHUNK, CH)
    didx2 = edge_index[1].reshape(NW, NCHUNK, CH)
    z128 = jnp.zeros((N, H2), jnp.float32)
    ones128 = jnp.ones((N, H2), jnp.float32)

    dsrc = _segsum(ones128, didx2, sidx2, z128)
    ddst = _segsum(ones128, sidx2, didx2, z128)
    ns, nd = _t_norms(dsrc, ddst)

    h0 = _t0(feats, ns, p['gcn0_W'])
    m0 = _segsum(h0, sidx2, didx2, z128)
    h1m = _t1(m0, nd, ns, p['gcn0_b'].reshape(1, HID), p['gcn1_W'])
    m1 = _segsum(h1m, sidx2, didx2, z128)
    h2m = _t2(m1, nd, ns, p['gcn1_b'].reshape(1, HID))
    m2 = _segsum(h2m, sidx2, didx2, z128)
    hcur = _t3(m2, p['gcn2_W'], nd, p['gcn2_b'].reshape(1, H2))

    for i in range(N_LAYER):
        alf = p[f'blk{i}_gat_al'].reshape(1, H2)
        arf = p[f'blk{i}_gat_ar'].reshape(1, H2)
        bgat = p[f'blk{i}_gat_b'].reshape(1, H2)
        hw, el, erp = _t4(hcur, p[f'blk{i}_gat_W'], alf, arf)
        cmax = _t5(el, erp)
        hg, erg = _gat_gather(hw, erp, sidx2, didx2)
        ee, hs = _t6(hg, erg, cmax, alf)
        esum = _escatter(ee, didx2, z128)
        pre = _escatter(hs, didx2, z128)
        h1b, h1ns = _t7(pre, esum, bgat, ns)
        mgc = _segsum(h1ns, sidx2, didx2, z128)
        hcur = _t8(mgc, nd, p[f'blk{i}_gc_W'], p[f'blk{i}_gc_b'].reshape(1, H4),
                   h1b, hcur,
                   p[f'blk{i}_ln_g'].reshape(1, H2), p[f'blk{i}_ln_b'].reshape(1, H2),
                   p['gate_w_r1'], p['gate_w_r2'], p['gate_b_r'].reshape(1, H2),
                   p['gate_w_z1'], p['gate_w_z2'], p['gate_b_z'].reshape(1, H2),
                   p['gate_w_h1'], p['gate_w_h2'], p['gate_b_h'].reshape(1, H2))

    return _t9(hcur)


# speedup vs baseline: 15.5477x; 1.0508x over previous
import functools

import jax
import jax.numpy as jnp
from jax import lax
from jax.experimental import pallas as pl
from jax.experimental.pallas import tpu as pltpu
from jax.experimental.pallas import tpu_sc as plsc

N = 10000
E = 320000
D_IN = 128
HID = 64
H2 = 128
H4 = 256
HEADS = 4
HEAD_DIM = 32
N_LAYER = 2

NC = 2
NS = 16
NW = NC * NS
CH = 80
EPW = E // NW
NCHUNK = EPW // CH
RPT = 1000
NZT = N // RPT

@functools.cache
def _mesh():
    return plsc.VectorSubcoreMesh(core_axis_name="c", subcore_axis_name="s")


def _segsum_body(x_hbm, sidx_hbm, didx_hbm, zeros_hbm, out_hbm,
                 sidx_v, didx_v, rows_v, acc_sh, sem):
    cid = lax.axis_index("c")
    sid = lax.axis_index("s")
    wid = sid * NC + cid
    r0 = sid * RPT

    @pl.when(sid < NZT)
    def _():
        pltpu.sync_copy(zeros_hbm.at[pl.ds(r0, RPT)],
                        acc_sh.at[pl.ds(r0, RPT)])

    pltpu.sync_copy(sidx_hbm.at[wid], sidx_v)
    pltpu.sync_copy(didx_hbm.at[wid], didx_v)
    plsc.subcore_barrier()

    def body(j, carry):
        pltpu.async_copy(x_hbm.at[sidx_v.at[j]], rows_v, sem).wait()
        pltpu.sync_copy(rows_v, acc_sh.at[didx_v.at[j]], add=True)
        return carry

    lax.fori_loop(0, NCHUNK, body, 0)
    plsc.subcore_barrier()

    @pl.when(sid < NZT)
    def _():
        pltpu.sync_copy(acc_sh.at[pl.ds(r0, RPT)],
                        out_hbm.at[cid, pl.ds(r0, RPT)])


@functools.cache
def _segsum_k():
    return pl.kernel(
        _segsum_body,
        out_type=jax.ShapeDtypeStruct((NC, N, H2), jnp.float32),
        mesh=_mesh(),
        scratch_types=[
            pltpu.VMEM((NCHUNK, CH), jnp.int32),
            pltpu.VMEM((NCHUNK, CH), jnp.int32),
            pltpu.VMEM((CH, H2), jnp.float32),
            pltpu.VMEM_SHARED((N, H2), jnp.float32),
            pltpu.SemaphoreType.DMA,
        ],
    )


def _segsum(*args):
    return _segsum_k()(*args)


def _gat_gather_body(hw_hbm, erp_hbm, sidx_hbm, didx_hbm,
                     hg_hbm, erg_hbm,
                     sidx_v, didx_v, rh_v, rb_v, sem0, sem1):
    cid = lax.axis_index("c")
    sid = lax.axis_index("s")
    wid = sid * NC + cid
    pltpu.sync_copy(sidx_hbm.at[wid], sidx_v)
    pltpu.sync_copy(didx_hbm.at[wid], didx_v)

    def body(j, carry):
        base = wid * EPW + j * CH
        a = pltpu.async_copy(hw_hbm.at[sidx_v.at[j]], rh_v, sem0)
        b = pltpu.async_copy(erp_hbm.at[didx_v.at[j]], rb_v, sem1)
        a.wait()
        b.wait()
        pltpu.sync_copy(rh_v, hg_hbm.at[pl.ds(base, CH)])
        pltpu.sync_copy(rb_v, erg_hbm.at[pl.ds(base, CH)])
        return carry

    lax.fori_loop(0, NCHUNK, body, 0)


@functools.cache
def _gat_gather_k():
    return pl.kernel(
        _gat_gather_body,
        out_type=(jax.ShapeDtypeStruct((E, H2), jnp.float32),
                  jax.ShapeDtypeStruct((E, H2), jnp.float32)),
        mesh=_mesh(),
        scratch_types=[
            pltpu.VMEM((NCHUNK, CH), jnp.int32),
            pltpu.VMEM((NCHUNK, CH), jnp.int32),
            pltpu.VMEM((CH, H2), jnp.float32),
            pltpu.VMEM((CH, H2), jnp.float32),
            pltpu.SemaphoreType.DMA,
            pltpu.SemaphoreType.DMA,
        ],
    )


def _gat_gather(*args):
    return _gat_gather_k()(*args)


def _escatter_body(vals_hbm, didx_hbm, zeros_hbm, out_hbm,
                   didx_v, rows_v, acc_sh):
    cid = lax.axis_index("c")
    sid = lax.axis_index("s")
    wid = sid * NC + cid
    r0 = sid * RPT

    @pl.when(sid < NZT)
    def _():
        pltpu.sync_copy(zeros_hbm.at[pl.ds(r0, RPT)],
                        acc_sh.at[pl.ds(r0, RPT)])

    pltpu.sync_copy(didx_hbm.at[wid], didx_v)
    plsc.subcore_barrier()

    def body(j, carry):
        base = wid * EPW + j * CH
        pltpu.sync_copy(vals_hbm.at[pl.ds(base, CH)], rows_v)
        pltpu.sync_copy(rows_v, acc_sh.at[didx_v.at[j]], add=True)
        return carry

    lax.fori_loop(0, NCHUNK, body, 0)
    plsc.subcore_barrier()

    @pl.when(sid < NZT)
    def _():
        pltpu.sync_copy(acc_sh.at[pl.ds(r0, RPT)],
                        out_hbm.at[cid, pl.ds(r0, RPT)])


@functools.cache
def _escatter_k():
    return pl.kernel(
        _escatter_body,
        out_type=jax.ShapeDtypeStruct((NC, N, H2), jnp.float32),
        mesh=_mesh(),
        scratch_types=[
            pltpu.VMEM((NCHUNK, CH), jnp.int32),
            pltpu.VMEM((CH, H2), jnp.float32),
            pltpu.VMEM_SHARED((N, H2), jnp.float32),
        ],
    )


def _escatter(*args):
    return _escatter_k()(*args)


def _count_body(ones_hbm, idx_hbm, zeros_hbm, out_hbm,
                idx_v, rows_v, acc_sh):
    cid = lax.axis_index("c")
    sid = lax.axis_index("s")
    wid = sid * NC + cid
    r0 = sid * RPT

    @pl.when(sid < NZT)
    def _():
        pltpu.sync_copy(zeros_hbm.at[pl.ds(r0, RPT)],
                        acc_sh.at[pl.ds(r0, RPT)])

    pltpu.sync_copy(ones_hbm, rows_v)
    pltpu.sync_copy(idx_hbm.at[wid], idx_v)
    plsc.subcore_barrier()

    def body(j, carry):
        pltpu.sync_copy(rows_v, acc_sh.at[idx_v.at[j]], add=True)
        return carry

    lax.fori_loop(0, NCHUNK, body, 0)
    plsc.subcore_barrier()

    @pl.when(sid < NZT)
    def _():
        pltpu.sync_copy(acc_sh.at[pl.ds(r0, RPT)],
                        out_hbm.at[cid, pl.ds(r0, RPT)])


@functools.cache
def _count_k():
    return pl.kernel(
        _count_body,
        out_type=jax.ShapeDtypeStruct((NC, N, H2), jnp.float32),
        mesh=_mesh(),
        scratch_types=[
            pltpu.VMEM((NCHUNK, CH), jnp.int32),
            pltpu.VMEM((CH, H2), jnp.float32),
            pltpu.VMEM_SHARED((N, H2), jnp.float32),
        ],
    )


def _count(*args):
    return _count_k()(*args)


RB = 2000
GRID_N = N // RB
EB = 5000
GRID_E = E // EB

PADW = H2 - HID


def _full(shape):
    return pl.BlockSpec(shape, lambda i: tuple(0 for _ in shape))


def _rows(shape):
    return pl.BlockSpec(shape, lambda i: (i,) + tuple(0 for _ in shape[1:]))


def _pad128(x):
    return jnp.concatenate(
        [x, jnp.zeros((x.shape[0], PADW), jnp.float32)], axis=1)


def _norms_body(ds_ref, dd_ref, ns_ref, nd_ref):
    ds = (ds_ref[0] + ds_ref[1])[:, :1]
    dd = (dd_ref[0] + dd_ref[1])[:, :1]
    ns_ref[...] = lax.rsqrt(jnp.maximum(ds, 1.0))
    nd_ref[...] = lax.rsqrt(jnp.maximum(dd, 1.0))


def _t_norms(dsrc, ddst):
    return pl.pallas_call(
        _norms_body,
        out_shape=(jax.ShapeDtypeStruct((N, 1), jnp.float32),
                   jax.ShapeDtypeStruct((N, 1), jnp.float32)),
        grid=(GRID_N,),
        in_specs=[pl.BlockSpec((NC, RB, H2), lambda i: (0, i, 0)),
                  pl.BlockSpec((NC, RB, H2), lambda i: (0, i, 0))],
        out_specs=(_rows((RB, 1)), _rows((RB, 1))),
    )(dsrc, ddst)


def _t0_body(x_ref, ns_ref, w_ref, o_ref):
    h = jnp.dot(x_ref[...] * ns_ref[...], w_ref[...],
                preferred_element_type=jnp.float32)
    o_ref[...] = _pad128(h)


def _t0(x, ns, w):
    return pl.pallas_call(
        _t0_body,
        out_shape=jax.ShapeDtypeStruct((N, H2), jnp.float32),
        grid=(GRID_N,),
        in_specs=[_rows((RB, x.shape[1])), _rows((RB, 1)),
                  _full(w.shape)],
        out_specs=_rows((RB, H2)),
    )(x, ns, w)


def _t1_body(m_ref, nd_ref, ns_ref, b_ref, w_ref, o_ref):
    m = (m_ref[0] + m_ref[1])[:, :HID]
    x = jnp.maximum(m * nd_ref[...] + b_ref[...], 0.0)
    h = jnp.dot(x * ns_ref[...], w_ref[...],
                preferred_element_type=jnp.float32)
    o_ref[...] = _pad128(h)


def _t1(m, nd, ns, b, w):
    return pl.pallas_call(
        _t1_body,
        out_shape=jax.ShapeDtypeStruct((N, H2), jnp.float32),
        grid=(GRID_N,),
        in_specs=[pl.BlockSpec((NC, RB, H2), lambda i: (0, i, 0)),
                  _rows((RB, 1)), _rows((RB, 1)), _full((1, HID)),
                  _full(w.shape)],
        out_specs=_rows((RB, H2)),
    )(m, nd, ns, b, w)


def _t2_body(m_ref, nd_ref, ns_ref, b_ref, o_ref):
    m = (m_ref[0] + m_ref[1])[:, :HID]
    x = jnp.maximum(m * nd_ref[...] + b_ref[...], 0.0)
    o_ref[...] = _pad128(x * ns_ref[...])


def _t2(m, nd, ns, b):
    return pl.pallas_call(
        _t2_body,
        out_shape=jax.ShapeDtypeStruct((N, H2), jnp.float32),
        grid=(GRID_N,),
        in_specs=[pl.BlockSpec((NC, RB, H2), lambda i: (0, i, 0)),
                  _rows((RB, 1)), _rows((RB, 1)), _full((1, HID))],
        out_specs=_rows((RB, H2)),
    )(m, nd, ns, b)


def _t3_body(m_ref, w_ref, nd_ref, b_ref, o_ref):
    m = (m_ref[0] + m_ref[1])[:, :HID]
    mm = jnp.dot(m, w_ref[...], preferred_element_type=jnp.float32)
    o_ref[...] = jnp.maximum(mm * nd_ref[...] + b_ref[...], 0.0)


def _t3(m, w, nd, b):
    return pl.pallas_call(
        _t3_body,
        out_shape=jax.ShapeDtypeStruct((N, w.shape[1]), jnp.float32),
        grid=(GRID_N,),
        in_specs=[pl.BlockSpec((NC, RB, H2), lambda i: (0, i, 0)),
                  _full(w.shape), _rows((RB, 1)), _full((1, w.shape[1]))],
        out_specs=_rows((RB, w.shape[1])),
    )(m, w, nd, b)


def _t4_body(h_ref, w_ref, al_ref, ar_ref, hw_ref, el_ref, erp_ref):
    hw = jnp.dot(h_ref[...], w_ref[...], preferred_element_type=jnp.float32)
    hw_ref[...] = hw
    el_ref[...] = (hw * al_ref[...]).reshape(RB, HEADS, HEAD_DIM).sum(-1)
    er = (hw * ar_ref[...]).reshape(RB, HEADS, HEAD_DIM).sum(-1)
    erp_ref[...] = jnp.concatenate(
        [er, jnp.zeros((RB, H2 - HEADS), jnp.float32)], axis=1)


def _t4(h, w, alf, arf):
    return pl.pallas_call(
        _t4_body,
        out_shape=(jax.ShapeDtypeStruct((N, H2), jnp.float32),
                   jax.ShapeDtypeStruct((N, HEADS), jnp.float32),
                   jax.ShapeDtypeStruct((N, H2), jnp.float32)),
        grid=(GRID_N,),
        in_specs=[_rows((RB, H2)), _full((H2, H2)),
                  _full((1, H2)), _full((1, H2))],
        out_specs=(_rows((RB, H2)), _rows((RB, HEADS)), _rows((RB, H2))),
    )(h, w, alf, arf)


def _t5_body(el_ref, erp_ref, c_ref):
    c = jnp.max(el_ref[...], axis=0) + jnp.max(erp_ref[...][:, :HEADS], axis=0)
    c_ref[...] = jnp.maximum(c, 0.0)[None]


def _t5(el, erp):
    return pl.pallas_call(
        _t5_body,
        out_shape=jax.ShapeDtypeStruct((1, HEADS), jnp.float32),
        grid=(1,),
        in_specs=[_full((N, HEADS)), _full((N, H2))],
        out_specs=_full((1, HEADS)),
    )(el, erp)


def _t6_body(hg_ref, erg_ref, c_ref, al_ref, ee_ref, hs_ref):
    hg = hg_ref[...]
    elg = (hg * al_ref[...]).reshape(EB, HEADS, HEAD_DIM).sum(-1)
    s = elg + erg_ref[...][:, :HEADS]
    e = jnp.maximum(s, 0.2 * s)
    ee = jnp.exp(e - c_ref[...])
    ee_ref[...] = jnp.concatenate(
        [ee, jnp.zeros((EB, H2 - HEADS), jnp.float32)], axis=1)
    ee128 = jnp.broadcast_to(ee[:, :, None], (EB, HEADS, HEAD_DIM))
    hs_ref[...] = hg * ee128.reshape(EB, H2)


def _t6(hg, erg, c, alf):
    return pl.pallas_call(
        _t6_body,
        out_shape=(jax.ShapeDtypeStruct((E, H2), jnp.float32),
                   jax.ShapeDtypeStruct((E, H2), jnp.float32)),
        grid=(GRID_E,),
        in_specs=[_rows((EB, H2)), _rows((EB, H2)),
                  _full((1, HEADS)), _full((1, H2))],
        out_specs=(_rows((EB, H2)), _rows((EB, H2))),
    )(hg, erg, c, alf)


def _t7_body(pre_ref, es_ref, b_ref, ns_ref, h1_ref, h1ns_ref):
    es = (es_ref[0] + es_ref[1])[:, :HEADS]
    den = jnp.where(es > 0.0, es, 1.0)
    den128 = jnp.broadcast_to(den[:, :, None], (RB, HEADS, HEAD_DIM))
    h1 = jnp.maximum((pre_ref[0] + pre_ref[1]) / den128.reshape(RB, H2)
                     + b_ref[...], 0.0)
    h1_ref[...] = h1
    h1ns_ref[...] = h1 * ns_ref[...]


def _t7(pre, esum, b, ns):
    return pl.pallas_call(
        _t7_body,
        out_shape=(jax.ShapeDtypeStruct((N, H2), jnp.float32),
                   jax.ShapeDtypeStruct((N, H2), jnp.float32)),
        grid=(GRID_N,),
        in_specs=[pl.BlockSpec((NC, RB, H2), lambda i: (0, i, 0)),
                  pl.BlockSpec((NC, RB, H2), lambda i: (0, i, 0)),
                  _full((1, H2)), _rows((RB, 1))],
        out_specs=(_rows((RB, H2)), _rows((RB, H2))),
    )(pre, esum, b, ns)


def _t8_body(m_ref, nd_ref, wgc_ref, bgc_ref, h1_ref, hp_ref,
             lng_ref, lnb_ref, wr1_ref, wr2_ref, br_ref,
             wz1_ref, wz2_ref, bz_ref, wh1_ref, wh2_ref, bh_ref, o_ref):
    mm = jnp.dot(m_ref[0] + m_ref[1], wgc_ref[...],
                 preferred_element_type=jnp.float32)
    h1_1 = jnp.maximum(mm * nd_ref[...] + bgc_ref[...], 0.0)
    a = h1_1[:, :H2]
    g = h1_1[:, H2:]
    glu = a * jax.nn.sigmoid(g)
    h2 = (h1_ref[...] + glu) * jnp.sqrt(jnp.float32(0.5))
    mu = jnp.mean(h2, axis=-1, keepdims=True)
    var = jnp.mean((h2 - mu) ** 2, axis=-1, keepdims=True)
    h2n = (h2 - mu) / jnp.sqrt(var + 1e-5) * lng_ref[...] + lnb_ref[...]
    hp = hp_ref[...]
    dot = functools.partial(jnp.dot, preferred_element_type=jnp.float32)
    r = jax.nn.sigmoid(dot(hp, wr1_ref[...]) + dot(h2n, wr2_ref[...])
                       + br_ref[...])
    z = jax.nn.sigmoid(dot(hp, wz1_ref[...]) + dot(h2n, wz2_ref[...])
                       + bz_ref[...])
    hc = jnp.tanh(dot(h2n, wh1_ref[...]) + dot(r * hp, wh2_ref[...])
                  + bh_ref[...])
    o_ref[...] = z * hp + (1.0 - z) * hc


def _t8(m, nd, wgc, bgc, h1, hprev, lng, lnb, wr1, wr2, br,
        wz1, wz2, bz, wh1, wh2, bh):
    return pl.pallas_call(
        _t8_body,
        out_shape=jax.ShapeDtypeStruct((N, H2), jnp.float32),
        grid=(GRID_N,),
        in_specs=[pl.BlockSpec((NC, RB, H2), lambda i: (0, i, 0)),
                  _rows((RB, 1)), _full((H2, H4)), _full((1, H4)),
                  _rows((RB, H2)), _rows((RB, H2)),
                  _full((1, H2)), _full((1, H2)),
                  _full((H2, H2)), _full((H2, H2)), _full((1, H2)),
                  _full((H2, H2)), _full((H2, H2)), _full((1, H2)),
                  _full((H2, H2)), _full((H2, H2)), _full((1, H2))],
        out_specs=_rows((RB, H2)),
    )(m, nd, wgc, bgc, h1, hprev, lng, lnb, wr1, wr2, br,
      wz1, wz2, bz, wh1, wh2, bh)


def _t9_body(h_ref, o_ref):
    o_ref[...] = jnp.mean(h_ref[...], axis=0, keepdims=True)


def _t9(h):
    return pl.pallas_call(
        _t9_body,
        out_shape=jax.ShapeDtypeStruct((1, H2), jnp.float32),
        grid=(1,),
        in_specs=[_full((N, H2))],
        out_specs=_full((1, H2)),
    )(h)


def kernel(feats, edge_index, params):
    p = params
    sidx2 = edge_index[0].reshape(NW, NCHUNK, CH)
    didx2 = edge_index[1].reshape(NW, NCHUNK, CH)
    z128 = jnp.zeros((N, H2), jnp.float32)
    ones_ch = jnp.ones((CH, H2), jnp.float32)

    dsrc = _count(ones_ch, sidx2, z128)
    ddst = _count(ones_ch, didx2, z128)
    ns, nd = _t_norms(dsrc, ddst)

    h0 = _t0(feats, ns, p['gcn0_W'])
    m0 = _segsum(h0, sidx2, didx2, z128)
    h1m = _t1(m0, nd, ns, p['gcn0_b'].reshape(1, HID), p['gcn1_W'])
    m1 = _segsum(h1m, sidx2, didx2, z128)
    h2m = _t2(m1, nd, ns, p['gcn1_b'].reshape(1, HID))
    m2 = _segsum(h2m, sidx2, didx2, z128)
    hcur = _t3(m2, p['gcn2_W'], nd, p['gcn2_b'].reshape(1, H2))

    for i in range(N_LAYER):
        alf = p[f'blk{i}_gat_al'].reshape(1, H2)
        arf = p[f'blk{i}_gat_ar'].reshape(1, H2)
        bgat = p[f'blk{i}_gat_b'].reshape(1, H2)
        hw, el, erp = _t4(hcur, p[f'blk{i}_gat_W'], alf, arf)
        cmax = _t5(el, erp)
        hg, erg = _gat_gather(hw, erp, sidx2, didx2)
        ee, hs = _t6(hg, erg, cmax, alf)
        esum = _escatter(ee, didx2, z128)
        pre = _escatter(hs, didx2, z128)
        h1b, h1ns = _t7(pre, esum, bgat, ns)
        mgc = _segsum(h1ns, sidx2, didx2, z128)
        hcur = _t8(mgc, nd, p[f'blk{i}_gc_W'], p[f'blk{i}_gc_b'].reshape(1, H4),
                   h1b, hcur,
                   p[f'blk{i}_ln_g'].reshape(1, H2), p[f'blk{i}_ln_b'].reshape(1, H2),
                   p['gate_w_r1'], p['gate_w_r2'], p['gate_b_r'].reshape(1, H2),
                   p['gate_w_z1'], p['gate_w_z2'], p['gate_b_z'].reshape(1, H2),
                   p['gate_w_h1'], p['gate_w_h2'], p['gate_b_h'].reshape(1, H2))

    return _t9(hcur)
